# baseline (speedup 1.0000x reference)
import functools
import math

import jax
import jax.numpy as jnp
from jax import lax
from jax.experimental import pallas as pl
from jax.experimental.pallas import tpu as pltpu
from jax.experimental.pallas import tpu_sc as plsc

B, S, D = 4096, 200, 64
N = B * S
NC, NS = 2, 16
NW = NC * NS
PER_W = N // NW
CHUNK = 128
NCHUNK = PER_W // CHUNK
NBUF = 4
ROUNDS = NCHUNK // NBUF
LANES = 16
VECS_PER_ROW = D // LANES
SCALE = math.sqrt(D)

_mesh = plsc.VectorSubcoreMesh(core_axis_name="c", subcore_axis_name="s")


@functools.partial(
    pl.kernel,
    mesh=_mesh,
    out_type=jax.ShapeDtypeStruct((N, D), jnp.float32),
    scratch_types=(
        [pltpu.VMEM((PER_W,), jnp.int32)]
        + [pltpu.VMEM((CHUNK, D), jnp.float32) for _ in range(NBUF)]
        + [pltpu.VMEM((CHUNK, D), jnp.float32) for _ in range(NBUF)]
        + [pltpu.SemaphoreType.DMA for _ in range(2 * NBUF)]
    ),
    compiler_params=pltpu.CompilerParams(use_tc_tiling_on_sc=False),
)
def _emb_lookup(x_hbm, table_hbm, out_hbm, idx_v, *scratch):
    rows = scratch[:NBUF]
    outs = scratch[NBUF:2 * NBUF]
    gsems = scratch[2 * NBUF:3 * NBUF]
    osems = scratch[3 * NBUF:4 * NBUF]

    wid = lax.axis_index("s") * NC + lax.axis_index("c")
    base = wid * PER_W

    pltpu.sync_copy(x_hbm.at[pl.ds(base, PER_W)], idx_v)

    def gather_desc(g, b):
        return pltpu.make_async_copy(
            table_hbm.at[idx_v.at[pl.ds(g * CHUNK, CHUNK)]],
            rows[b],
            gsems[b],
        )

    def out_desc(g, b):
        return pltpu.make_async_copy(
            outs[b],
            out_hbm.at[pl.ds(base + g * CHUNK, CHUNK)],
            osems[b],
        )

    for b in range(NBUF):
        gather_desc(b, b).start()

    def round_body(t, _):
        g0 = t * NBUF
        for b in range(NBUF):
            g = g0 + b
            gather_desc(g, b).wait()

            @pl.when(t > 0)
            def _wait_prev_out():
                out_desc(g - NBUF, b).wait()

            def scale_row(r, _):
                for v in range(VECS_PER_ROW):
                    sl = pl.ds(v * LANES, LANES)
                    outs[b][r, sl] = rows[b][r, sl] * SCALE
                return ()

            lax.fori_loop(0, CHUNK, scale_row, (), unroll=2)

            out_desc(g, b).start()

            @pl.when(t < ROUNDS - 1)
            def _prefetch_next():
                gather_desc(g + NBUF, b).start()
        return ()

    lax.fori_loop(0, ROUNDS, round_body, ())

    for b in range(NBUF):
        out_desc((ROUNDS - 1) * NBUF + b, b).wait()


def kernel(x, table):
    xf = x.reshape(-1).astype(jnp.int32)
    out = _emb_lookup(xf, table)
    return out.reshape(x.shape + (D,))
